# baseline (speedup 1.0000x reference)
import functools

import jax
import jax.numpy as jnp
from jax import lax
from jax.experimental import pallas as pl
from jax.experimental.pallas import tpu as pltpu
from jax.experimental.pallas import tpu_sc as plsc

B = 1024
INPUT_SIZE = 128
H = 128
STACK = 200

_NC = 1
_NS = 16
_NW = _NC * _NS
_BPW = B // _NW

@functools.cache
def _make_sc_gather():
    mesh = plsc.VectorSubcoreMesh(core_axis_name="c", subcore_axis_name="s")

    @functools.partial(
        pl.kernel,
        out_type=jax.ShapeDtypeStruct((2, B, H), jnp.float32),
        mesh=mesh,
        scratch_types=[
            pltpu.VMEM((_BPW,), jnp.int32),
            pltpu.VMEM((_BPW,), jnp.int32),
            pltpu.VMEM((2, _BPW, H), jnp.float32),
            pltpu.SemaphoreType.DMA,
            pltpu.SemaphoreType.DMA,
        ],
    )
    def _sc_gather(htab, ctab, pos_hbm, out_hc, pos_v, idx_v, rows_v, sem_h, sem_c):
        cid = lax.axis_index("c")
        base = lax.axis_index("s") * _BPW
        pltpu.sync_copy(pos_hbm.at[pl.ds(base, _BPW)], pos_v)
        for j in range(_BPW // 16):
            sl = pl.ds(j * 16, 16)
            idx_v[sl] = pos_v[sl] * B + (base + j * 16 + lax.iota(jnp.int32, 16))

        @pl.when(cid == 0)
        def _():
            pltpu.async_copy(htab.at[idx_v], rows_v.at[0], sem_h).wait()
            pltpu.sync_copy(rows_v.at[0], out_hc.at[0].at[pl.ds(base, _BPW)])

        @pl.when(cid == 1)
        def _():
            pltpu.async_copy(ctab.at[idx_v], rows_v.at[0], sem_c).wait()
            pltpu.sync_copy(rows_v.at[0], out_hc.at[1].at[pl.ds(base, _BPW)])

    return _sc_gather


_DN_T = (((1,), (1,)), ((), ()))


def _lstm_body(x_ref, hc_ref, wih_ref, whh_ref, bih_ref, bhh_ref, op_ref, oh_ref, oc_ref):
    h = hc_ref[0]
    c = hc_ref[1]
    gates = (
        lax.dot_general(x_ref[...], wih_ref[...], _DN_T, preferred_element_type=jnp.float32)
        + lax.dot_general(h, whh_ref[...], _DN_T, preferred_element_type=jnp.float32)
        + bih_ref[...]
        + bhh_ref[...]
    )
    i = jax.nn.sigmoid(gates[:, 0:H])
    f = jax.nn.sigmoid(gates[:, H : 2 * H])
    g = jnp.tanh(gates[:, 2 * H : 3 * H])
    o = jax.nn.sigmoid(gates[:, 3 * H : 4 * H])
    c_new = f * c + i * g
    h_new = o * jnp.tanh(c_new)
    push = op_ref[...] > 0
    oh_ref[...] = jnp.where(push, h_new, h)
    oc_ref[...] = jnp.where(push, c_new, c)


_lstm_call = pl.pallas_call(
    _lstm_body,
    out_shape=(
        jax.ShapeDtypeStruct((B, H), jnp.float32),
        jax.ShapeDtypeStruct((B, H), jnp.float32),
    ),
)


def kernel(input, op, pos, hidden_stack, cell_stack, W_ih, W_hh, b_ih, b_hh):
    htab = hidden_stack.reshape((STACK + 1) * B, H)
    ctab = cell_stack.reshape((STACK + 1) * B, H)
    cur_hc = _make_sc_gather()(htab, ctab, pos)
    out_h, out_c = _lstm_call(
        input,
        cur_hc,
        W_ih[0],
        W_hh[0],
        b_ih.reshape(1, 4 * H),
        b_hh.reshape(1, 4 * H),
        op.reshape(B, 1),
    )
    return out_h, out_c


# speedup vs baseline: 1.0197x; 1.0197x over previous
import functools

import jax
import jax.numpy as jnp
from jax import lax
from jax.experimental import pallas as pl
from jax.experimental.pallas import tpu as pltpu
from jax.experimental.pallas import tpu_sc as plsc

B = 1024
INPUT_SIZE = 128
H = 128
STACK = 200

_NC = 1
_NS = 16
_NW = _NC * _NS
_BPW = B // _NW

@functools.cache
def _make_sc_gather():
    mesh = plsc.VectorSubcoreMesh(core_axis_name="c", subcore_axis_name="s", num_cores=1)

    @functools.partial(
        pl.kernel,
        out_type=jax.ShapeDtypeStruct((2, B, H), jnp.float32),
        mesh=mesh,
        scratch_types=[
            pltpu.VMEM((_BPW,), jnp.int32),
            pltpu.VMEM((_BPW,), jnp.int32),
            pltpu.VMEM((2, _BPW, H), jnp.float32),
            pltpu.SemaphoreType.DMA,
            pltpu.SemaphoreType.DMA,
        ],
    )
    def _sc_gather(htab, ctab, pos_hbm, out_hc, pos_v, idx_v, rows_v, sem_h, sem_c):
        wid = lax.axis_index("s") * _NC + lax.axis_index("c")
        base = wid * _BPW
        pltpu.sync_copy(pos_hbm.at[pl.ds(base, _BPW)], pos_v)
        for j in range(_BPW // 16):
            sl = pl.ds(j * 16, 16)
            idx_v[sl] = pos_v[sl] * B + (base + j * 16 + lax.iota(jnp.int32, 16))
        cp_h = pltpu.async_copy(htab.at[idx_v], rows_v.at[0], sem_h)
        cp_c = pltpu.async_copy(ctab.at[idx_v], rows_v.at[1], sem_c)
        cp_h.wait()
        out_h = pltpu.async_copy(rows_v.at[0], out_hc.at[0].at[pl.ds(base, _BPW)], sem_h)
        cp_c.wait()
        out_c = pltpu.async_copy(rows_v.at[1], out_hc.at[1].at[pl.ds(base, _BPW)], sem_c)
        out_h.wait()
        out_c.wait()

    return _sc_gather


_DN_T = (((1,), (1,)), ((), ()))


def _lstm_body(x_ref, hc_hbm, wih_ref, whh_ref, bih_ref, bhh_ref, op_ref, oh_ref, oc_ref,
               hc_vmem, sem):
    cp = pltpu.make_async_copy(hc_hbm, hc_vmem, sem)
    cp.start()
    xg = (
        lax.dot_general(x_ref[...], wih_ref[...], _DN_T, preferred_element_type=jnp.float32)
        + bih_ref[...]
        + bhh_ref[...]
    )
    cp.wait()
    h = hc_vmem[0]
    c = hc_vmem[1]
    gates = xg + lax.dot_general(h, whh_ref[...], _DN_T, preferred_element_type=jnp.float32)
    i = jax.nn.sigmoid(gates[:, 0:H])
    f = jax.nn.sigmoid(gates[:, H : 2 * H])
    g = jnp.tanh(gates[:, 2 * H : 3 * H])
    o = jax.nn.sigmoid(gates[:, 3 * H : 4 * H])
    c_new = f * c + i * g
    h_new = o * jnp.tanh(c_new)
    push = op_ref[...] > 0
    oh_ref[...] = jnp.where(push, h_new, h)
    oc_ref[...] = jnp.where(push, c_new, c)


_lstm_call = pl.pallas_call(
    _lstm_body,
    in_specs=[
        pl.BlockSpec(memory_space=pltpu.VMEM),
        pl.BlockSpec(memory_space=pl.ANY),
        pl.BlockSpec(memory_space=pltpu.VMEM),
        pl.BlockSpec(memory_space=pltpu.VMEM),
        pl.BlockSpec(memory_space=pltpu.VMEM),
        pl.BlockSpec(memory_space=pltpu.VMEM),
        pl.BlockSpec(memory_space=pltpu.VMEM),
    ],
    scratch_shapes=[
        pltpu.VMEM((2, B, H), jnp.float32),
        pltpu.SemaphoreType.DMA,
    ],
    out_shape=(
        jax.ShapeDtypeStruct((B, H), jnp.float32),
        jax.ShapeDtypeStruct((B, H), jnp.float32),
    ),
)


def kernel(input, op, pos, hidden_stack, cell_stack, W_ih, W_hh, b_ih, b_hh):
    htab = hidden_stack.reshape((STACK + 1) * B, H)
    ctab = cell_stack.reshape((STACK + 1) * B, H)
    cur_hc = _make_sc_gather()(htab, ctab, pos)
    out_h, out_c = _lstm_call(
        input,
        cur_hc,
        W_ih[0],
        W_hh[0],
        b_ih.reshape(1, 4 * H),
        b_hh.reshape(1, 4 * H),
        op.reshape(B, 1),
    )
    return out_h, out_c


# speedup vs baseline: 1.0520x; 1.0317x over previous
import functools

import jax
import jax.numpy as jnp
from jax import lax
from jax.experimental import pallas as pl
from jax.experimental.pallas import tpu as pltpu
from jax.experimental.pallas import tpu_sc as plsc

B = 1024
INPUT_SIZE = 128
H = 128
STACK = 200

_NC = 1
_NS = 16
_NW = _NC * _NS
_BPW = B // _NW

@functools.cache
def _make_sc_gather():
    mesh = plsc.VectorSubcoreMesh(core_axis_name="c", subcore_axis_name="s", num_cores=1)

    @functools.partial(
        pl.kernel,
        out_type=jax.ShapeDtypeStruct((2, B, H), jnp.float32),
        mesh=mesh,
        scratch_types=[
            pltpu.VMEM((_BPW,), jnp.int32),
            pltpu.VMEM((_BPW,), jnp.int32),
            pltpu.VMEM((2, _BPW, H), jnp.float32),
            pltpu.SemaphoreType.DMA,
            pltpu.SemaphoreType.DMA,
        ],
    )
    def _sc_gather(htab, ctab, pos_hbm, out_hc, pos_v, idx_v, rows_v, sem_h, sem_c):
        wid = lax.axis_index("s") * _NC + lax.axis_index("c")
        base = wid * _BPW
        pltpu.sync_copy(pos_hbm.at[pl.ds(base, _BPW)], pos_v)
        for j in range(_BPW // 16):
            sl = pl.ds(j * 16, 16)
            idx_v[sl] = pos_v[sl] * B + (base + j * 16 + lax.iota(jnp.int32, 16))
        cp_h = pltpu.async_copy(htab.at[idx_v], rows_v.at[0], sem_h)
        cp_c = pltpu.async_copy(ctab.at[idx_v], rows_v.at[1], sem_c)
        cp_h.wait()
        out_h = pltpu.async_copy(rows_v.at[0], out_hc.at[0].at[pl.ds(base, _BPW)], sem_h)
        cp_c.wait()
        out_c = pltpu.async_copy(rows_v.at[1], out_hc.at[1].at[pl.ds(base, _BPW)], sem_c)
        out_h.wait()
        out_c.wait()

    return _sc_gather


_DN_T = (((1,), (1,)), ((), ()))


def _lstm_body(x_ref, hc_ref, wih_ref, whh_ref, bih_ref, bhh_ref, op_ref, oh_ref, oc_ref):
    h = hc_ref[0]
    c = hc_ref[1]
    gates = (
        lax.dot_general(x_ref[...], wih_ref[...], _DN_T, preferred_element_type=jnp.float32)
        + lax.dot_general(h, whh_ref[...], _DN_T, preferred_element_type=jnp.float32)
        + bih_ref[...]
        + bhh_ref[...]
    )
    i = jax.nn.sigmoid(gates[:, 0:H])
    f = jax.nn.sigmoid(gates[:, H : 2 * H])
    g = jnp.tanh(gates[:, 2 * H : 3 * H])
    o = jax.nn.sigmoid(gates[:, 3 * H : 4 * H])
    c_new = f * c + i * g
    h_new = o * jnp.tanh(c_new)
    push = op_ref[...] > 0
    oh_ref[...] = jnp.where(push, h_new, h)
    oc_ref[...] = jnp.where(push, c_new, c)


_lstm_call = pl.pallas_call(
    _lstm_body,
    out_shape=(
        jax.ShapeDtypeStruct((B, H), jnp.float32),
        jax.ShapeDtypeStruct((B, H), jnp.float32),
    ),
)


def kernel(input, op, pos, hidden_stack, cell_stack, W_ih, W_hh, b_ih, b_hh):
    htab = hidden_stack.reshape((STACK + 1) * B, H)
    ctab = cell_stack.reshape((STACK + 1) * B, H)
    cur_hc = _make_sc_gather()(htab, ctab, pos)
    out_h, out_c = _lstm_call(
        input,
        cur_hc,
        W_ih[0],
        W_hh[0],
        b_ih.reshape(1, 4 * H),
        b_hh.reshape(1, 4 * H),
        op.reshape(B, 1),
    )
    return out_h, out_c
